# baseline (speedup 1.0000x reference)
import functools

import jax
import jax.numpy as jnp
from jax import lax
from jax.experimental import pallas as pl
from jax.experimental.pallas import tpu as pltpu
from jax.experimental.pallas import tpu_sc as plsc

NC = 2
NS = 16
NW = NC * NS
LANES = 16


def _emb_body(T, D, rows_per_worker, idx_hbm, tab_hbm, pos_hbm, out_hbm,
              idx_v, rows_v, pos_v, sem):
  wid = lax.axis_index("s") * NC + lax.axis_index("c")
  base_row = wid * rows_per_worker

  pltpu.sync_copy(pos_hbm, pos_v)

  def row_body(r, carry):
    g = base_row + r
    pltpu.sync_copy(idx_hbm.at[pl.ds(g * T, T)], idx_v)
    cp1 = pltpu.async_copy(tab_hbm.at[idx_v.at[pl.ds(0, 104)]],
                           rows_v.at[pl.ds(0, 104)], sem)
    cp2 = pltpu.async_copy(tab_hbm.at[idx_v.at[pl.ds(104, 96)]],
                           rows_v.at[pl.ds(104, 96)], sem)
    cp1.wait()
    cp2.wait()

    def add_body(t, carry2):
      for c0 in range(0, D, LANES):
        rows_v[t, pl.ds(c0, LANES)] = (rows_v[t, pl.ds(c0, LANES)]
                                       + pos_v[t, pl.ds(c0, LANES)])
      return carry2

    lax.fori_loop(0, T, add_body, 0, unroll=2)
    pltpu.sync_copy(rows_v, out_hbm.at[pl.ds(g * T, T)])
    return carry

  lax.fori_loop(0, rows_per_worker, row_body, 0)


def kernel(inputs, token_table, pos_table):
  B, T = inputs.shape
  V, D = token_table.shape
  rows_per_worker = B // NW

  mesh = plsc.VectorSubcoreMesh(core_axis_name="c", subcore_axis_name="s",
                                num_cores=NC, num_subcores=NS)
  emb = pl.kernel(
      functools.partial(_emb_body, T, D, rows_per_worker),
      out_type=jax.ShapeDtypeStruct((B * T, D), jnp.float32),
      mesh=mesh,
      compiler_params=pltpu.CompilerParams(use_tc_tiling_on_sc=False),
      scratch_types=[
          pltpu.VMEM((T,), jnp.int32),
          pltpu.VMEM((T, D), jnp.float32),
          pltpu.VMEM((T, D), jnp.float32),
          pltpu.SemaphoreType.DMA,
      ],
  )
  idx_flat = inputs.reshape(-1).astype(jnp.int32)
  out = emb(idx_flat, token_table, pos_table)
  return out.reshape(B, T, D)


# speedup vs baseline: 1.4932x; 1.4932x over previous
import functools

import jax
import jax.numpy as jnp
from jax import lax
from jax.experimental import pallas as pl
from jax.experimental.pallas import tpu as pltpu
from jax.experimental.pallas import tpu_sc as plsc

NC = 2
NS = 16
NW = NC * NS
LANES = 16
K = 4


def _chunks(n):
  out = []
  off = 0
  while off < n:
    sz = min(128, n - off)
    out.append((off, sz))
    off += sz
  return out


def _emb_body(T, D, rpw, idx_hbm, tab_hbm, pos_hbm, out_hbm,
              idx_v, rows_v, pos_v,
              isem0, isem1, gsem0, gsem1, ssem0, ssem1):
  wid = lax.axis_index("s") * NC + lax.axis_index("c")
  base_row = wid * rpw
  GT = K * T
  ngroups = rpw // K
  npairs = ngroups // 2
  isems = (isem0, isem1)
  gsems = (gsem0, gsem1)
  ssems = (ssem0, ssem1)

  def tok_base(g):
    return (base_row + g * K) * T

  def idx_copy(b, g):
    return pltpu.make_async_copy(idx_hbm.at[pl.ds(tok_base(g), GT)],
                                 idx_v.at[b], isems[b])

  def gather_copies(b):
    return [pltpu.make_async_copy(tab_hbm.at[idx_v.at[b, pl.ds(off, sz)]],
                                  rows_v.at[b, pl.ds(off, sz)], gsems[b])
            for off, sz in _chunks(GT)]

  def store_copy(b, g):
    return pltpu.make_async_copy(rows_v.at[b],
                                 out_hbm.at[pl.ds(tok_base(g), GT)], ssems[b])

  def add_pos(b):
    def tbody(t, carry):
      for c in range(0, D, LANES):
        pv = pos_v[t, pl.ds(c, LANES)]
        for k in range(K):
          r = k * T + t
          rows_v[b, r, pl.ds(c, LANES)] = rows_v[b, r, pl.ds(c, LANES)] + pv
      return carry
    lax.fori_loop(0, T, tbody, 0, unroll=2)

  def process(g, b, first, fire_next, prefetch_idx):
    nb = 1 - b
    if fire_next:
      idx_copy(nb, g).wait()
      if not first:
        store_copy(nb, g).wait()
      for c in gather_copies(nb):
        c.start()
    for c in gather_copies(b):
      c.wait()
    if prefetch_idx:
      idx_copy(b, g + 2).start()
    add_pos(b)
    store_copy(b, g).start()

  pltpu.sync_copy(pos_hbm, pos_v)
  idx_copy(0, 0).start()
  idx_copy(0, 0).wait()
  for c in gather_copies(0):
    c.start()
  idx_copy(1, 1).start()

  def pair(i, first_pair, last_pair):
    g = 2 * i
    process(g, 0, first=first_pair, fire_next=True,
            prefetch_idx=not last_pair)
    process(g + 1, 1, first=False, fire_next=not last_pair,
            prefetch_idx=not last_pair)

  pair(0, True, False)
  lax.fori_loop(1, npairs - 1, lambda i, c: (pair(i, False, False), c)[1], 0)
  pair(npairs - 1, False, True)

  store_copy(0, 0).wait()
  store_copy(1, 0).wait()


def kernel(inputs, token_table, pos_table):
  B, T = inputs.shape
  V, D = token_table.shape
  rpw = B // NW
  GT = K * T

  mesh = plsc.VectorSubcoreMesh(core_axis_name="c", subcore_axis_name="s",
                                num_cores=NC, num_subcores=NS)
  emb = pl.kernel(
      functools.partial(_emb_body, T, D, rpw),
      out_type=jax.ShapeDtypeStruct((B * T, D), jnp.float32),
      mesh=mesh,
      compiler_params=pltpu.CompilerParams(use_tc_tiling_on_sc=False),
      scratch_types=[
          pltpu.VMEM((2, GT), jnp.int32),
          pltpu.VMEM((2, GT, D), jnp.float32),
          pltpu.VMEM((T, D), jnp.float32),
          pltpu.SemaphoreType.DMA,
          pltpu.SemaphoreType.DMA,
          pltpu.SemaphoreType.DMA,
          pltpu.SemaphoreType.DMA,
          pltpu.SemaphoreType.DMA,
          pltpu.SemaphoreType.DMA,
      ],
  )
  idx_flat = inputs.reshape(-1).astype(jnp.int32)
  out = emb(idx_flat, token_table, pos_table)
  return out.reshape(B, T, D)


# speedup vs baseline: 1.4937x; 1.0003x over previous
import functools

import jax
import jax.numpy as jnp
from jax import lax
from jax.experimental import pallas as pl
from jax.experimental.pallas import tpu as pltpu
from jax.experimental.pallas import tpu_sc as plsc

NC = 2
NS = 16
NW = NC * NS
LANES = 16
K = 4


def _row_chunks(T):
  half = (T // 2 + 7) // 8 * 8
  return ((0, half), (half, T - half))


def _emb_body(T, D, rpw, idx_hbm, tab_hbm, pos_hbm, out_hbm,
              idx_v, rows_v, pos_v,
              isem0, isem1, gsem0, gsem1, ssem0, ssem1):
  wid = lax.axis_index("s") * NC + lax.axis_index("c")
  base_row = wid * rpw
  ngroups = rpw // K
  npairs = ngroups // 2
  isems = (isem0, isem1)
  gsems = (gsem0, gsem1)
  ssems = (ssem0, ssem1)

  def idx_copy(b, g):
    return pltpu.make_async_copy(idx_hbm.at[pl.ds(base_row + g * K, K)],
                                 idx_v.at[b], isems[b])

  def gather_copies(b):
    cps = []
    for k in range(K):
      for off, sz in _row_chunks(T):
        cps.append(pltpu.make_async_copy(
            tab_hbm.at[idx_v.at[b, k, pl.ds(off, sz)]],
            rows_v.at[b, k, pl.ds(off, sz)], gsems[b]))
    return cps

  def store_copy(b, g):
    return pltpu.make_async_copy(rows_v.at[b],
                                 out_hbm.at[pl.ds(base_row + g * K, K)],
                                 ssems[b])

  def add_pos(b):
    def tbody(t, carry):
      for c in range(0, D, LANES):
        pv = pos_v[t, pl.ds(c, LANES)]
        for k in range(K):
          rows_v[b, k, t, pl.ds(c, LANES)] = (
              rows_v[b, k, t, pl.ds(c, LANES)] + pv)
      return carry
    lax.fori_loop(0, T, tbody, 0, unroll=2)

  def process(g, b, first, fire_next, prefetch_idx):
    nb = 1 - b
    if fire_next:
      idx_copy(nb, g).wait()
      if not first:
        store_copy(nb, g).wait()
      for c in gather_copies(nb):
        c.start()
    for c in gather_copies(b):
      c.wait()
    if prefetch_idx:
      idx_copy(b, g + 2).start()
    add_pos(b)
    store_copy(b, g).start()

  pltpu.sync_copy(pos_hbm, pos_v)
  idx_copy(0, 0).start()
  idx_copy(0, 0).wait()
  for c in gather_copies(0):
    c.start()
  idx_copy(1, 1).start()

  def pair(i, first_pair, last_pair):
    g = 2 * i
    process(g, 0, first=first_pair, fire_next=True,
            prefetch_idx=not last_pair)
    process(g + 1, 1, first=False, fire_next=not last_pair,
            prefetch_idx=not last_pair)

  pair(0, True, False)
  lax.fori_loop(1, npairs - 1, lambda i, c: (pair(i, False, False), c)[1], 0)
  pair(npairs - 1, False, True)

  store_copy(0, 0).wait()
  store_copy(1, 0).wait()


def kernel(inputs, token_table, pos_table):
  B, T = inputs.shape
  V, D = token_table.shape
  rpw = B // NW

  mesh = plsc.VectorSubcoreMesh(core_axis_name="c", subcore_axis_name="s",
                                num_cores=NC, num_subcores=NS)
  emb = pl.kernel(
      functools.partial(_emb_body, T, D, rpw),
      out_type=jax.ShapeDtypeStruct((B, T, D), jnp.float32),
      mesh=mesh,
      compiler_params=pltpu.CompilerParams(use_tc_tiling_on_sc=False),
      scratch_types=[
          pltpu.VMEM((2, K, T), jnp.int32),
          pltpu.VMEM((2, K, T, D), jnp.float32),
          pltpu.VMEM((T, D), jnp.float32),
          pltpu.SemaphoreType.DMA,
          pltpu.SemaphoreType.DMA,
          pltpu.SemaphoreType.DMA,
          pltpu.SemaphoreType.DMA,
          pltpu.SemaphoreType.DMA,
          pltpu.SemaphoreType.DMA,
      ],
  )
  return emb(inputs.astype(jnp.int32), token_table, pos_table)
